# Initial kernel scaffold; baseline (speedup 1.0000x reference)
import jax
import jax.numpy as jnp
from jax.experimental import pallas as pl


def kernel(x):
    raise NotImplementedError("write your pallas kernel here")


# speedup vs baseline: 13.4772x; 13.4772x over previous
import functools

import jax
import jax.numpy as jnp
from jax import lax
from jax.experimental import pallas as pl
from jax.experimental.pallas import tpu as pltpu

_K = 16
_BIG_I = 2**30


def _select_topk(v, i, k):
    last_v = jnp.full((1, v.shape[1]), jnp.inf, jnp.float32)
    last_i = jnp.full((1, v.shape[1]), -1, jnp.int32)
    vs, is_ = [], []
    for _ in range(k):
        allowed = (v < last_v) | ((v == last_v) & (i > last_i))
        vm = jnp.max(jnp.where(allowed, v, -jnp.inf), axis=0, keepdims=True)
        im = jnp.min(
            jnp.where(allowed & (v == vm), i, jnp.int32(_BIG_I)),
            axis=0,
            keepdims=True,
        )
        vs.append(vm)
        is_.append(im)
        last_v, last_i = vm, im
    return jnp.concatenate(vs, axis=0), jnp.concatenate(is_, axis=0)


def _kmax_body(x_ref, o_ref, run_v, run_i, *, ns, sb):
    sblk = pl.program_id(1)

    @pl.when(sblk == 0)
    def _init():
        run_v[...] = jnp.full((_K, 128), -jnp.inf, jnp.float32)
        run_i[...] = jnp.full((_K, 128), jnp.int32(_BIG_I), jnp.int32)

    blk = x_ref[0]
    iota = lax.broadcasted_iota(jnp.int32, blk.shape, 0) + sblk * sb
    bv, bi = _select_topk(blk, iota, _K)

    mv = jnp.concatenate([run_v[...], bv], axis=0)
    mi = jnp.concatenate([run_i[...], bi], axis=0)
    nv, ni = _select_topk(mv, mi, _K)
    run_v[...] = nv
    run_i[...] = ni

    @pl.when(sblk == ns - 1)
    def _finish():
        fv, fi = run_v[...], run_i[...]
        rank = jnp.sum(
            (fi[None, :, :] < fi[:, None, :]).astype(jnp.int32), axis=1
        )
        slot = lax.broadcasted_iota(jnp.int32, (_K, _K, 128), 0)
        onehot = rank[None, :, :] == slot
        out = jnp.sum(jnp.where(onehot, fv[None, :, :], 0.0), axis=1)
        o_ref[...] = out[None]


def kernel(x):
    b, s, d = x.shape
    assert d == 128
    sb = min(s, 4096)
    assert s % sb == 0
    ns = s // sb
    f = pl.pallas_call(
        functools.partial(_kmax_body, ns=ns, sb=sb),
        grid=(b, ns),
        in_specs=[pl.BlockSpec((1, sb, d), lambda i, j: (i, j, 0))],
        out_specs=pl.BlockSpec((1, _K, d), lambda i, j: (i, 0, 0)),
        out_shape=jax.ShapeDtypeStruct((b, _K, d), jnp.float32),
        scratch_shapes=[
            pltpu.VMEM((_K, d), jnp.float32),
            pltpu.VMEM((_K, d), jnp.int32),
        ],
        compiler_params=pltpu.CompilerParams(
            dimension_semantics=("arbitrary", "arbitrary"),
        ),
    )
    return f(x)


# speedup vs baseline: 19.5852x; 1.4532x over previous
import functools

import jax
import jax.numpy as jnp
from jax import lax
from jax.experimental import pallas as pl
from jax.experimental.pallas import tpu as pltpu
from jax.experimental.pallas import tpu_sc as plsc

_K = 16
_L = 16
_CS = 1024
_C = 1024
_BOOT = 64
_BIG = 2**30


def _sc_body(x_hbm, o_hbm, buf, cand_v, cand_i, heap_v, heap_i, tvec, obuf,
             *, nb, nd, units_per):
    li = lax.iota(jnp.int32, _L)
    wid = lax.axis_index("c") * 16 + lax.axis_index("s")

    def scan(lo, hi, s0):
        tv = tvec[...]

        def step(s, cnt):
            v = buf[s]
            m = v > tv
            addr = li * _C + cnt
            plsc.store_scatter(cand_v, [addr], v, mask=m)
            plsc.store_scatter(
                cand_i, [addr], jnp.full((_L,), s0, jnp.int32) + s, mask=m
            )
            return cnt + m.astype(jnp.int32)

        return lax.fori_loop(lo, hi, step, jnp.zeros((_L,), jnp.int32))

    def merges(cnt):
        def merge_j(j, _):
            cj = jnp.sum(jnp.where(li == j, cnt, 0))

            @pl.when(cj > 0)
            def _do():
                hv0 = heap_v[pl.ds(j * _K, _K)]
                hi0 = heap_i[pl.ds(j * _K, _K)]

                def mb(g, carry):
                    hv, hi = carry
                    base = j * _C + g * _L
                    kv = cand_v[pl.ds(base, _L)]
                    ki = cand_i[pl.ds(base, _L)]
                    valid = (g * _L + li) < cj
                    kv = jnp.where(valid, kv, -jnp.inf)
                    ki = jnp.where(valid, ki, _BIG)
                    kv, ki = plsc.sort_key_val(kv, ki)
                    rkv = lax.rev(kv, (0,))
                    rki = lax.rev(ki, (0,))
                    keep = (hv > rkv) | ((hv == rkv) & (hi < rki))
                    mv = jnp.where(keep, hv, rkv)
                    mi = jnp.where(keep, hi, rki)
                    return tuple(plsc.sort_key_val(mv, mi))

                nv = (cj + _L - 1) // _L
                hv1, hi1 = lax.fori_loop(0, nv, mb, (hv0, hi0))
                heap_v[pl.ds(j * _K, _K)] = hv1
                heap_i[pl.ds(j * _K, _K)] = hi1
                tmin = jnp.min(hv1)
                tvec[...] = jnp.where(li == j, tmin, tvec[...])

            return 0

        lax.fori_loop(0, _L, merge_j, 0)

    def unit_body(u, _):
        unit = wid * units_per + u
        b = unit // nd
        d0 = (unit % nd) * _L

        def init_j(j, _):
            heap_v[pl.ds(j * _K, _K)] = jnp.full((_K,), -jnp.inf, jnp.float32)
            heap_i[pl.ds(j * _K, _K)] = jnp.full((_K,), _BIG, jnp.int32)
            return 0

        lax.fori_loop(0, _L, init_j, 0)
        tvec[...] = jnp.full((_L,), -jnp.inf, jnp.float32)

        def chunk_body(g, _):
            s0 = g * _CS
            pltpu.sync_copy(x_hbm.at[b, pl.ds(s0, _CS), pl.ds(d0, _L)], buf)

            @pl.when(g == 0)
            def _boot():
                merges(scan(0, _BOOT, s0))

            lo = jnp.where(g == 0, _BOOT, 0)
            merges(scan(lo, _CS, s0))
            return 0

        lax.fori_loop(0, nb, chunk_body, 0)

        def out_j(j, _):
            hv = heap_v[pl.ds(j * _K, _K)]
            hi = heap_i[pl.ds(j * _K, _K)]
            _, kv = plsc.sort_key_val(hi, hv)
            plsc.store_scatter(obuf, [li, jnp.full((_L,), j, jnp.int32)], kv)
            return 0

        lax.fori_loop(0, _L, out_j, 0)
        pltpu.sync_copy(obuf, o_hbm.at[b, :, pl.ds(d0, _L)])
        return 0

    lax.fori_loop(0, units_per, unit_body, 0)


def kernel(x):
    b, s, d = x.shape
    assert d % _L == 0 and s % _CS == 0
    nd = d // _L
    nb = s // _CS
    n_workers = 32
    units = b * nd
    assert units % n_workers == 0
    mesh = plsc.VectorSubcoreMesh(core_axis_name="c", subcore_axis_name="s")
    f = pl.kernel(
        functools.partial(
            _sc_body, nb=nb, nd=nd, units_per=units // n_workers
        ),
        out_type=jax.ShapeDtypeStruct((b, _K, d), jnp.float32),
        mesh=mesh,
        compiler_params=pltpu.CompilerParams(
            use_tc_tiling_on_sc=False, needs_layout_passes=False
        ),
        scratch_types=[
            pltpu.VMEM((_CS, _L), jnp.float32),
            pltpu.VMEM((_L * _C,), jnp.float32),
            pltpu.VMEM((_L * _C,), jnp.int32),
            pltpu.VMEM((_L * _K,), jnp.float32),
            pltpu.VMEM((_L * _K,), jnp.int32),
            pltpu.VMEM((_L,), jnp.float32),
            pltpu.VMEM((_K, _L), jnp.float32),
        ],
    )
    return f(x)


# speedup vs baseline: 36.4609x; 1.8617x over previous
import functools

import jax
import jax.numpy as jnp
from jax import lax
from jax.experimental import pallas as pl
from jax.experimental.pallas import tpu as pltpu
from jax.experimental.pallas import tpu_sc as plsc

_K = 16
_L = 16
_CS = 1024
_C = 1024
_U = 8
_BOOT = 16
_BIG = 2**30


def _sc_body(x_hbm, o_hbm, buf0, buf1, cand_v, cand_i, heap_v, heap_i, tvec,
             obuf, sem0, sem1, *, nb, nd, units_per):
    li = lax.iota(jnp.int32, _L)
    li_c = li * _C
    wid = lax.axis_index("c") * 16 + lax.axis_index("s")

    def scan(buf, lo, hi, s0, cnt0):
        tv = tvec[...]

        def step(g, cnt):
            base = g * _U
            vs = [buf[base + i] for i in range(_U)]
            m01 = jnp.maximum(vs[0], vs[1])
            m23 = jnp.maximum(vs[2], vs[3])
            m45 = jnp.maximum(vs[4], vs[5])
            m67 = jnp.maximum(vs[6], vs[7])
            mx = jnp.maximum(jnp.maximum(m01, m23), jnp.maximum(m45, m67))

            def app(cnt):
                for i in range(_U):
                    m = vs[i] > tv
                    addr = li_c + cnt
                    plsc.store_scatter(cand_v, [addr], vs[i], mask=m)
                    plsc.store_scatter(
                        cand_i,
                        [addr],
                        jnp.full((_L,), s0 + i, jnp.int32) + base,
                        mask=m,
                    )
                    cnt = cnt + m.astype(jnp.int32)
                return cnt

            return lax.cond(jnp.any(mx > tv), app, lambda c: c, cnt)

        return lax.fori_loop(lo // _U, hi // _U, step, cnt0)

    def merges(cnt):
        def merge_j(j, _):
            cj = jnp.sum(jnp.where(li == j, cnt, 0))

            @pl.when(cj > 0)
            def _do():
                hv0 = heap_v[pl.ds(j * _K, _K)]
                hi0 = heap_i[pl.ds(j * _K, _K)]

                def mb(g, carry):
                    hv, hi = carry
                    base = j * _C + g * _L
                    kv = cand_v[pl.ds(base, _L)]
                    ki = cand_i[pl.ds(base, _L)]
                    valid = (g * _L + li) < cj
                    kv = jnp.where(valid, kv, -jnp.inf)
                    ki = jnp.where(valid, ki, _BIG)
                    kv, ki = plsc.sort_key_val(kv, ki)
                    rkv = lax.rev(kv, (0,))
                    rki = lax.rev(ki, (0,))
                    keep = (hv > rkv) | ((hv == rkv) & (hi < rki))
                    mv = jnp.where(keep, hv, rkv)
                    mi = jnp.where(keep, hi, rki)
                    return tuple(plsc.sort_key_val(mv, mi))

                nv = (cj + _L - 1) // _L
                hv1, hi1 = lax.fori_loop(0, nv, mb, (hv0, hi0))
                heap_v[pl.ds(j * _K, _K)] = hv1
                heap_i[pl.ds(j * _K, _K)] = hi1
                tmin = jnp.min(hv1)
                tvec[...] = jnp.where(li == j, tmin, tvec[...])

            return 0

        lax.fori_loop(0, _L, merge_j, 0)

    def unit_body(u, _):
        unit = wid * units_per + u
        b = unit // nd
        d0 = (unit % nd) * _L

        def init_j(j, _):
            heap_v[pl.ds(j * _K, _K)] = jnp.full((_K,), -jnp.inf, jnp.float32)
            heap_i[pl.ds(j * _K, _K)] = jnp.full((_K,), _BIG, jnp.int32)
            return 0

        lax.fori_loop(0, _L, init_j, 0)
        tvec[...] = jnp.full((_L,), -jnp.inf, jnp.float32)

        def dma(g, buf, sem):
            return pltpu.make_async_copy(
                x_hbm.at[b, pl.ds(g * _CS, _CS), pl.ds(d0, _L)], buf, sem
            )

        def process(buf, g):
            s0 = g * _CS

            def boot(_):
                merges(scan(buf, 0, _BOOT, s0, jnp.zeros((_L,), jnp.int32)))
                return scan(buf, _BOOT, _CS, s0, jnp.zeros((_L,), jnp.int32))

            def plain(_):
                return scan(buf, 0, _CS, s0, jnp.zeros((_L,), jnp.int32))

            merges(lax.cond(g == 0, boot, plain, 0))

        dma(0, buf0, sem0).start()

        def pair_body(p, _):
            g0 = 2 * p
            dma(g0, buf0, sem0).wait()
            dma(g0 + 1, buf1, sem1).start()
            process(buf0, g0)
            dma(g0 + 1, buf1, sem1).wait()

            @pl.when(g0 + 2 < nb)
            def _next():
                dma(g0 + 2, buf0, sem0).start()

            process(buf1, g0 + 1)
            return 0

        lax.fori_loop(0, nb // 2, pair_body, 0)

        def out_j(j, _):
            hv = heap_v[pl.ds(j * _K, _K)]
            hi = heap_i[pl.ds(j * _K, _K)]
            _, kv = plsc.sort_key_val(hi, hv)
            plsc.store_scatter(obuf, [li, jnp.full((_L,), j, jnp.int32)], kv)
            return 0

        lax.fori_loop(0, _L, out_j, 0)
        pltpu.sync_copy(obuf, o_hbm.at[b, :, pl.ds(d0, _L)])
        return 0

    lax.fori_loop(0, units_per, unit_body, 0)


def kernel(x):
    b, s, d = x.shape
    assert d % _L == 0 and s % _CS == 0 and (s // _CS) % 2 == 0
    nd = d // _L
    nb = s // _CS
    n_workers = 32
    units = b * nd
    assert units % n_workers == 0
    mesh = plsc.VectorSubcoreMesh(core_axis_name="c", subcore_axis_name="s")
    f = pl.kernel(
        functools.partial(
            _sc_body, nb=nb, nd=nd, units_per=units // n_workers
        ),
        out_type=jax.ShapeDtypeStruct((b, _K, d), jnp.float32),
        mesh=mesh,
        compiler_params=pltpu.CompilerParams(
            use_tc_tiling_on_sc=False, needs_layout_passes=False
        ),
        scratch_types=[
            pltpu.VMEM((_CS, _L), jnp.float32),
            pltpu.VMEM((_CS, _L), jnp.float32),
            pltpu.VMEM((_L * _C,), jnp.float32),
            pltpu.VMEM((_L * _C,), jnp.int32),
            pltpu.VMEM((_L * _K,), jnp.float32),
            pltpu.VMEM((_L * _K,), jnp.int32),
            pltpu.VMEM((_L,), jnp.float32),
            pltpu.VMEM((_K, _L), jnp.float32),
            pltpu.SemaphoreType.DMA,
            pltpu.SemaphoreType.DMA,
        ],
    )
    return f(x)


# speedup vs baseline: 37.2905x; 1.0228x over previous
import functools

import jax
import jax.numpy as jnp
from jax import lax
from jax.experimental import pallas as pl
from jax.experimental.pallas import tpu as pltpu
from jax.experimental.pallas import tpu_sc as plsc

_K = 16
_L = 16
_CS = 1024
_C = 1024
_U = 32
_SUB = 8
_BOOT = 32
_BIG = 2**30


def _sc_body(x_hbm, o_hbm, buf0, buf1, cand_v, cand_i, heap_v, heap_i, tvec,
             obuf, sem0, sem1, *, nb, nd, units_per):
    li = lax.iota(jnp.int32, _L)
    li_c = li * _C
    wid = lax.axis_index("c") * 16 + lax.axis_index("s")

    def scan(buf, lo, hi, s0, cnt0):
        tv = tvec[...]

        def _tree_max(ws):
            while len(ws) > 1:
                ws = [
                    jnp.maximum(ws[2 * t], ws[2 * t + 1])
                    for t in range(len(ws) // 2)
                ]
            return ws[0]

        def step(g, cnt):
            base = g * _U
            subs = [
                _tree_max([buf[base + _SUB * t + i] for i in range(_SUB)])
                for t in range(_U // _SUB)
            ]
            mx = _tree_max(list(subs))

            def app(cnt):
                for t in range(_U // _SUB):

                    def app_t(cnt, t=t):
                        for i in range(_SUB):
                            r = _SUB * t + i
                            v = buf[base + r]
                            m = v > tv
                            addr = li_c + cnt
                            plsc.store_scatter(cand_v, [addr], v, mask=m)
                            plsc.store_scatter(
                                cand_i,
                                [addr],
                                jnp.full((_L,), s0 + r, jnp.int32) + base,
                                mask=m,
                            )
                            cnt = cnt + m.astype(jnp.int32)
                        return cnt

                    cnt = lax.cond(
                        jnp.any(subs[t] > tv), app_t, lambda c: c, cnt
                    )
                return cnt

            return lax.cond(jnp.any(mx > tv), app, lambda c: c, cnt)

        return lax.fori_loop(lo // _U, hi // _U, step, cnt0)

    def merges(cnt):
        def merge_j(j, _):
            cj = jnp.sum(jnp.where(li == j, cnt, 0))

            @pl.when(cj > 0)
            def _do():
                hv0 = heap_v[pl.ds(j * _K, _K)]
                hi0 = heap_i[pl.ds(j * _K, _K)]

                def mb(g, carry):
                    hv, hi = carry
                    base = j * _C + g * _L
                    kv = cand_v[pl.ds(base, _L)]
                    ki = cand_i[pl.ds(base, _L)]
                    valid = (g * _L + li) < cj
                    kv = jnp.where(valid, kv, -jnp.inf)
                    ki = jnp.where(valid, ki, _BIG)
                    kv, ki = plsc.sort_key_val(kv, ki)
                    rkv = lax.rev(kv, (0,))
                    rki = lax.rev(ki, (0,))
                    keep = (hv > rkv) | ((hv == rkv) & (hi < rki))
                    mv = jnp.where(keep, hv, rkv)
                    mi = jnp.where(keep, hi, rki)
                    return tuple(plsc.sort_key_val(mv, mi))

                nv = (cj + _L - 1) // _L
                hv1, hi1 = lax.fori_loop(0, nv, mb, (hv0, hi0))
                heap_v[pl.ds(j * _K, _K)] = hv1
                heap_i[pl.ds(j * _K, _K)] = hi1
                tmin = jnp.min(hv1)
                tvec[...] = jnp.where(li == j, tmin, tvec[...])

            return 0

        lax.fori_loop(0, _L, merge_j, 0)

    def unit_body(u, _):
        unit = wid * units_per + u
        b = unit // nd
        d0 = (unit % nd) * _L

        def init_j(j, _):
            heap_v[pl.ds(j * _K, _K)] = jnp.full((_K,), -jnp.inf, jnp.float32)
            heap_i[pl.ds(j * _K, _K)] = jnp.full((_K,), _BIG, jnp.int32)
            return 0

        lax.fori_loop(0, _L, init_j, 0)
        tvec[...] = jnp.full((_L,), -jnp.inf, jnp.float32)

        def dma(g, buf, sem):
            return pltpu.make_async_copy(
                x_hbm.at[b, pl.ds(g * _CS, _CS), pl.ds(d0, _L)], buf, sem
            )

        def process(buf, g):
            s0 = g * _CS

            def boot(_):
                merges(scan(buf, 0, _BOOT, s0, jnp.zeros((_L,), jnp.int32)))
                return scan(buf, _BOOT, _CS, s0, jnp.zeros((_L,), jnp.int32))

            def plain(_):
                return scan(buf, 0, _CS, s0, jnp.zeros((_L,), jnp.int32))

            merges(lax.cond(g == 0, boot, plain, 0))

        dma(0, buf0, sem0).start()

        def pair_body(p, _):
            g0 = 2 * p
            dma(g0, buf0, sem0).wait()
            dma(g0 + 1, buf1, sem1).start()
            process(buf0, g0)
            dma(g0 + 1, buf1, sem1).wait()

            @pl.when(g0 + 2 < nb)
            def _next():
                dma(g0 + 2, buf0, sem0).start()

            process(buf1, g0 + 1)
            return 0

        lax.fori_loop(0, nb // 2, pair_body, 0)

        def out_j(j, _):
            hv = heap_v[pl.ds(j * _K, _K)]
            hi = heap_i[pl.ds(j * _K, _K)]
            _, kv = plsc.sort_key_val(hi, hv)
            plsc.store_scatter(obuf, [li, jnp.full((_L,), j, jnp.int32)], kv)
            return 0

        lax.fori_loop(0, _L, out_j, 0)
        pltpu.sync_copy(obuf, o_hbm.at[b, :, pl.ds(d0, _L)])
        return 0

    lax.fori_loop(0, units_per, unit_body, 0)


def kernel(x):
    b, s, d = x.shape
    assert d % _L == 0 and s % _CS == 0 and (s // _CS) % 2 == 0
    nd = d // _L
    nb = s // _CS
    n_workers = 32
    units = b * nd
    assert units % n_workers == 0
    mesh = plsc.VectorSubcoreMesh(core_axis_name="c", subcore_axis_name="s")
    f = pl.kernel(
        functools.partial(
            _sc_body, nb=nb, nd=nd, units_per=units // n_workers
        ),
        out_type=jax.ShapeDtypeStruct((b, _K, d), jnp.float32),
        mesh=mesh,
        compiler_params=pltpu.CompilerParams(
            use_tc_tiling_on_sc=False, needs_layout_passes=False
        ),
        scratch_types=[
            pltpu.VMEM((_CS, _L), jnp.float32),
            pltpu.VMEM((_CS, _L), jnp.float32),
            pltpu.VMEM((_L * _C,), jnp.float32),
            pltpu.VMEM((_L * _C,), jnp.int32),
            pltpu.VMEM((_L * _K,), jnp.float32),
            pltpu.VMEM((_L * _K,), jnp.int32),
            pltpu.VMEM((_L,), jnp.float32),
            pltpu.VMEM((_K, _L), jnp.float32),
            pltpu.SemaphoreType.DMA,
            pltpu.SemaphoreType.DMA,
        ],
    )
    return f(x)


# speedup vs baseline: 37.4765x; 1.0050x over previous
import functools

import jax
import jax.numpy as jnp
from jax import lax
from jax.experimental import pallas as pl
from jax.experimental.pallas import tpu as pltpu
from jax.experimental.pallas import tpu_sc as plsc

_K = 16
_L = 16
_CS = 1024
_C = 1024
_U = 32
_SUB = 8
_BOOT = 32
_BIG = 2**30


def _sc_body(x_hbm, o_hbm, buf0, buf1, cand_v, cand_i, heap_v, heap_i, tvec,
             obuf, sem0, sem1, *, nb, nd, units_per):
    li = lax.iota(jnp.int32, _L)
    wid = lax.axis_index("c") * 16 + lax.axis_index("s")

    def scan(buf, lo, hi, s0, cnt0):
        tv = tvec[...]

        def _tree_max(ws):
            while len(ws) > 1:
                ws = [
                    jnp.maximum(ws[2 * t], ws[2 * t + 1])
                    for t in range(len(ws) // 2)
                ]
            return ws[0]

        def step(g, cnt):
            base = g * _U
            subs = [
                _tree_max([buf[base + _SUB * t + i] for i in range(_SUB)])
                for t in range(_U // _SUB)
            ]
            mx = _tree_max(list(subs))

            def app(cnt):
                for t in range(_U // _SUB):

                    def app_t(cnt, t=t):
                        for i in range(_SUB):
                            r = _SUB * t + i
                            v = buf[base + r]
                            m = v > tv
                            addr = cnt * _L + li
                            plsc.store_scatter(cand_v, [addr], v, mask=m)
                            plsc.store_scatter(
                                cand_i,
                                [addr],
                                jnp.full((_L,), s0 + r, jnp.int32) + base,
                                mask=m,
                            )
                            cnt = cnt + m.astype(jnp.int32)
                        return cnt

                    cnt = lax.cond(
                        jnp.any(subs[t] > tv), app_t, lambda c: c, cnt
                    )
                return cnt

            return lax.cond(jnp.any(mx > tv), app, lambda c: c, cnt)

        return lax.fori_loop(lo // _U, hi // _U, step, cnt0)

    def merges(cnt):
        mc = jnp.max(cnt)

        @pl.when(mc > 0)
        def _do():
            def round_r(r, _):
                act = r < cnt
                cv = cand_v[pl.ds(r * _L, _L)]
                ci = cand_i[pl.ds(r * _L, _L)]
                cv = jnp.where(act, cv, -jnp.inf)
                ci = jnp.where(act, ci, _BIG)
                hq_v = heap_v[pl.ds(0, _L)]
                hq_i = heap_i[pl.ds(0, _L)]
                for q in range(_K):
                    up = cv > hq_v
                    t_v = jnp.where(up, cv, hq_v)
                    t_i = jnp.where(up, ci, hq_i)
                    if q == _K - 1:
                        heap_v[pl.ds(q * _L, _L)] = t_v
                        heap_i[pl.ds(q * _L, _L)] = t_i
                    else:
                        hn_v = heap_v[pl.ds((q + 1) * _L, _L)]
                        hn_i = heap_i[pl.ds((q + 1) * _L, _L)]
                        keep_t = (t_v < hn_v) | ((t_v == hn_v) & (t_i > hn_i))
                        heap_v[pl.ds(q * _L, _L)] = jnp.where(
                            keep_t, t_v, hn_v
                        )
                        heap_i[pl.ds(q * _L, _L)] = jnp.where(
                            keep_t, t_i, hn_i
                        )
                        hq_v, hq_i = hn_v, hn_i
                return 0

            lax.fori_loop(0, mc, round_r, 0)
            tvec[...] = heap_v[pl.ds(0, _L)]

    def unit_body(u, _):
        unit = wid * units_per + u
        b = unit // nd
        d0 = (unit % nd) * _L

        for q in range(_K):
            heap_v[pl.ds(q * _L, _L)] = jnp.full((_L,), -jnp.inf, jnp.float32)
            heap_i[pl.ds(q * _L, _L)] = jnp.full((_L,), _BIG, jnp.int32)
        tvec[...] = jnp.full((_L,), -jnp.inf, jnp.float32)

        def dma(g, buf, sem):
            return pltpu.make_async_copy(
                x_hbm.at[b, pl.ds(g * _CS, _CS), pl.ds(d0, _L)], buf, sem
            )

        def process(buf, g):
            s0 = g * _CS

            def boot(_):
                merges(scan(buf, 0, _BOOT, s0, jnp.zeros((_L,), jnp.int32)))
                return scan(buf, _BOOT, _CS, s0, jnp.zeros((_L,), jnp.int32))

            def plain(_):
                return scan(buf, 0, _CS, s0, jnp.zeros((_L,), jnp.int32))

            merges(lax.cond(g == 0, boot, plain, 0))

        dma(0, buf0, sem0).start()

        def pair_body(p, _):
            g0 = 2 * p
            dma(g0, buf0, sem0).wait()
            dma(g0 + 1, buf1, sem1).start()
            process(buf0, g0)
            dma(g0 + 1, buf1, sem1).wait()

            @pl.when(g0 + 2 < nb)
            def _next():
                dma(g0 + 2, buf0, sem0).start()

            process(buf1, g0 + 1)
            return 0

        lax.fori_loop(0, nb // 2, pair_body, 0)

        hv = [heap_v[pl.ds(q * _L, _L)] for q in range(_K)]
        hi = [heap_i[pl.ds(q * _L, _L)] for q in range(_K)]
        for row in range(_K):
            wv, wi = hv[0], hi[0]
            for q in range(1, _K):
                take = hi[q] < wi
                wv = jnp.where(take, hv[q], wv)
                wi = jnp.where(take, hi[q], wi)
            obuf[row] = wv
            if row < _K - 1:
                for q in range(_K):
                    used = hi[q] == wi
                    hi[q] = jnp.where(used, _BIG, hi[q])

        pltpu.sync_copy(obuf, o_hbm.at[b, :, pl.ds(d0, _L)])
        return 0

    lax.fori_loop(0, units_per, unit_body, 0)


def kernel(x):
    b, s, d = x.shape
    assert d % _L == 0 and s % _CS == 0 and (s // _CS) % 2 == 0
    nd = d // _L
    nb = s // _CS
    n_workers = 32
    units = b * nd
    assert units % n_workers == 0
    mesh = plsc.VectorSubcoreMesh(core_axis_name="c", subcore_axis_name="s")
    f = pl.kernel(
        functools.partial(
            _sc_body, nb=nb, nd=nd, units_per=units // n_workers
        ),
        out_type=jax.ShapeDtypeStruct((b, _K, d), jnp.float32),
        mesh=mesh,
        compiler_params=pltpu.CompilerParams(
            use_tc_tiling_on_sc=False, needs_layout_passes=False
        ),
        scratch_types=[
            pltpu.VMEM((_CS, _L), jnp.float32),
            pltpu.VMEM((_CS, _L), jnp.float32),
            pltpu.VMEM((_C * _L,), jnp.float32),
            pltpu.VMEM((_C * _L,), jnp.int32),
            pltpu.VMEM((_K * _L,), jnp.float32),
            pltpu.VMEM((_K * _L,), jnp.int32),
            pltpu.VMEM((_L,), jnp.float32),
            pltpu.VMEM((_K, _L), jnp.float32),
            pltpu.SemaphoreType.DMA,
            pltpu.SemaphoreType.DMA,
        ],
    )
    return f(x)
